# speedup vs baseline: 2.9753x; 2.9753x over previous
import functools

import jax
import jax.numpy as jnp
from jax import lax
from jax.experimental import pallas as pl
from jax.experimental.pallas import tpu as pltpu
from jax.experimental.pallas import tpu_sc as plsc

_NC = 2
_NS = 16
_NW = _NC * _NS

_CHUNK = 128


@functools.lru_cache(maxsize=None)
def _build_gather(n_flat: int, vocab: int, d: int):
    per_w = n_flat // _NW
    n_chunks = per_w // _CHUNK

    mesh = plsc.VectorSubcoreMesh(core_axis_name="c", subcore_axis_name="s")

    @functools.partial(
        pl.kernel,
        mesh=mesh,
        out_type=jax.ShapeDtypeStruct((n_flat, d), jnp.float32),
        scratch_types=[
            pltpu.VMEM((n_chunks, _CHUNK), jnp.int32),
            pltpu.VMEM((_CHUNK, d), jnp.float32),
            pltpu.SemaphoreType.DMA,
        ],
    )
    def gather_kernel(table_hbm, idx_hbm, out_hbm, idx_v, rows_v, sem):
        wid = lax.axis_index("s") * _NC + lax.axis_index("c")
        base = wid * per_w
        pltpu.sync_copy(idx_hbm.at[wid], idx_v)

        def step(i, carry):
            pltpu.async_copy(table_hbm.at[idx_v.at[i]], rows_v, sem).wait()
            pltpu.sync_copy(
                rows_v, out_hbm.at[pl.ds(base + i * _CHUNK, _CHUNK)]
            )
            return carry

        lax.fori_loop(0, n_chunks, step, 0)

    return gather_kernel


@jax.jit
def kernel(x, emb_matrix):
    b, h = x.shape
    vocab, d = emb_matrix.shape
    n_flat = b * h
    idx = x.reshape(_NW, n_flat // (_NW * _CHUNK), _CHUNK)
    out = _build_gather(n_flat, vocab, d)(emb_matrix, idx)
    return out.reshape(b, h, d)


# speedup vs baseline: 3.3220x; 1.1165x over previous
import functools

import jax
import jax.numpy as jnp
from jax import lax
from jax.experimental import pallas as pl
from jax.experimental.pallas import tpu as pltpu
from jax.experimental.pallas import tpu_sc as plsc

_NC = 2
_NS = 16
_NW = _NC * _NS

_CHUNK = 64
_RING = 5


@functools.lru_cache(maxsize=None)
def _build_gather(n_flat: int, vocab: int, d: int):
    per_w = n_flat // _NW
    n_chunks = per_w // _CHUNK
    n_groups = n_chunks // _RING
    half = n_groups // 2
    assert n_chunks == _RING * n_groups and n_groups == 2 * half

    mesh = plsc.VectorSubcoreMesh(core_axis_name="c", subcore_axis_name="s")
    n_sem = 2 * _RING

    @functools.partial(
        pl.kernel,
        mesh=mesh,
        out_type=jax.ShapeDtypeStruct((n_flat, d), jnp.float32),
        scratch_types=[
            pltpu.VMEM((n_chunks, _CHUNK), jnp.int32),
            pltpu.VMEM((n_sem, _CHUNK, d), jnp.float32),
        ]
        + [pltpu.SemaphoreType.DMA] * (2 * n_sem),
    )
    def gather_kernel(table_hbm, idx_hbm, out_hbm, idx_v, rows_v, *sems):
        gsems = sems[:n_sem]
        wsems = sems[n_sem:]
        wid = lax.axis_index("s") * _NC + lax.axis_index("c")
        base = wid * per_w
        pltpu.sync_copy(idx_hbm.at[wid], idx_v)

        def g_src(i):
            return table_hbm.at[idx_v.at[i]]

        def out_dst(i):
            return out_hbm.at[pl.ds(base + i * _CHUNK, _CHUNK)]

        for b in range(_RING):
            pltpu.async_copy(g_src(b), rows_v.at[b], gsems[b])

        def outer(gg, carry):
            for p in range(2):
                g = 2 * gg + p
                for b in range(_RING):
                    s = p * _RING + b
                    sn = (1 - p) * _RING + b
                    i = g * _RING + b
                    pltpu.make_async_copy(g_src(i), rows_v.at[s], gsems[s]).wait()
                    pltpu.async_copy(rows_v.at[s], out_dst(i), wsems[s])

                    def prefetch(i=i, s=s, sn=sn):
                        pltpu.make_async_copy(
                            rows_v.at[sn], out_dst(i - _RING), wsems[sn]
                        ).wait()
                        pltpu.async_copy(g_src(i + _RING), rows_v.at[sn], gsems[sn])

                    if p == 0:
                        @pl.when(gg > 0)
                        def _(i=i, s=s, sn=sn):
                            pltpu.make_async_copy(
                                rows_v.at[sn], out_dst(i - _RING), wsems[sn]
                            ).wait()

                        pltpu.async_copy(g_src(i + _RING), rows_v.at[sn], gsems[sn])
                    else:
                        pl.when(gg < half - 1)(prefetch)
            return carry

        lax.fori_loop(0, half, outer, 0)

        for b in range(_RING):
            i0 = (n_groups - 2) * _RING + b
            i1 = (n_groups - 1) * _RING + b
            pltpu.make_async_copy(rows_v.at[b], out_dst(i0), wsems[b]).wait()
            pltpu.make_async_copy(
                rows_v.at[_RING + b], out_dst(i1), wsems[_RING + b]
            ).wait()

    return gather_kernel


@jax.jit
def kernel(x, emb_matrix):
    b, h = x.shape
    vocab, d = emb_matrix.shape
    n_flat = b * h
    idx = x.reshape(_NW, n_flat // (_NW * _CHUNK), _CHUNK)
    out = _build_gather(n_flat, vocab, d)(emb_matrix, idx)
    return out.reshape(b, h, d)


# speedup vs baseline: 5.0827x; 1.5300x over previous
import functools

import jax
import jax.numpy as jnp
from jax import lax
from jax.experimental import pallas as pl
from jax.experimental.pallas import tpu as pltpu
from jax.experimental.pallas import tpu_sc as plsc

_NC = 2
_NS = 16
_NW = _NC * _NS

_RING = 4

_SUB = 8


@functools.lru_cache(maxsize=None)
def _build_gather(batch: int, hist: int, vocab: int, d: int):
    per_w = batch // _NW
    hist_pad = -(-hist // _SUB) * _SUB
    n_groups = per_w // _RING
    half = n_groups // 2
    assert per_w == _RING * n_groups and n_groups == 2 * half

    mesh = plsc.VectorSubcoreMesh(core_axis_name="c", subcore_axis_name="s")
    n_sem = 2 * _RING

    @functools.partial(
        pl.kernel,
        mesh=mesh,
        out_type=jax.ShapeDtypeStruct((batch * hist_pad, d), jnp.float32),
        scratch_types=[
            pltpu.VMEM((per_w, hist), jnp.int32),
            pltpu.VMEM((n_sem, hist_pad, d), jnp.float32),
        ]
        + [pltpu.SemaphoreType.DMA] * (2 * n_sem),
    )
    def gather_kernel(table_hbm, idx_hbm, out_hbm, idx_v, rows_v, *sems):
        gsems = sems[:n_sem]
        wsems = sems[n_sem:]
        wid = lax.axis_index("s") * _NC + lax.axis_index("c")
        base = wid * per_w
        pltpu.sync_copy(idx_hbm.at[wid], idx_v)

        def g_src(i):
            return table_hbm.at[idx_v.at[i]]

        def out_dst(i):
            return out_hbm.at[pl.ds((base + i) * hist_pad, hist_pad)]

        def g_dst(s):
            return rows_v.at[s, pl.ds(0, hist)]

        for b in range(_RING):
            pltpu.async_copy(g_src(b), g_dst(b), gsems[b])

        def outer(gg, carry):
            for p in range(2):
                g = 2 * gg + p
                for b in range(_RING):
                    s = p * _RING + b
                    sn = (1 - p) * _RING + b
                    i = g * _RING + b
                    pltpu.make_async_copy(g_src(i), g_dst(s), gsems[s]).wait()
                    pltpu.async_copy(rows_v.at[s], out_dst(i), wsems[s])

                    def prefetch(i=i, sn=sn):
                        pltpu.make_async_copy(
                            rows_v.at[sn], out_dst(i - _RING), wsems[sn]
                        ).wait()
                        pltpu.async_copy(g_src(i + _RING), g_dst(sn), gsems[sn])

                    if p == 0:
                        @pl.when(gg > 0)
                        def _(i=i, sn=sn):
                            pltpu.make_async_copy(
                                rows_v.at[sn], out_dst(i - _RING), wsems[sn]
                            ).wait()

                        pltpu.async_copy(g_src(i + _RING), g_dst(sn), gsems[sn])
                    else:
                        pl.when(gg < half - 1)(prefetch)
            return carry

        lax.fori_loop(0, half, outer, 0)

        for b in range(_RING):
            i0 = (n_groups - 2) * _RING + b
            i1 = (n_groups - 1) * _RING + b
            pltpu.make_async_copy(rows_v.at[b], out_dst(i0), wsems[b]).wait()
            pltpu.make_async_copy(
                rows_v.at[_RING + b], out_dst(i1), wsems[_RING + b]
            ).wait()

    return gather_kernel


@jax.jit
def kernel(x, emb_matrix):
    b, h = x.shape
    vocab, d = emb_matrix.shape
    h_pad = -(-h // _SUB) * _SUB
    idx = x.reshape(_NW, b // _NW, h)
    out = _build_gather(b, h, vocab, d)(emb_matrix, idx)
    return out.reshape(b, h_pad, d)[:, :h, :]


# speedup vs baseline: 5.9587x; 1.1723x over previous
import functools

import jax
import jax.numpy as jnp
from jax import lax
from jax.experimental import pallas as pl
from jax.experimental.pallas import tpu as pltpu
from jax.experimental.pallas import tpu_sc as plsc

_NC = 2
_NS = 16
_NW = _NC * _NS

_RING = 4

_SUB = 8


@functools.lru_cache(maxsize=None)
def _build_gather(batch: int, hist: int, vocab: int, d: int):
    per_w = batch // _NW
    hist_pad = -(-hist // _SUB) * _SUB
    n_groups = per_w // _RING
    half = n_groups // 2
    assert per_w == _RING * n_groups and n_groups == 2 * half

    mesh = plsc.VectorSubcoreMesh(core_axis_name="c", subcore_axis_name="s")
    n_sem = 2 * _RING

    @functools.partial(
        pl.kernel,
        mesh=mesh,
        out_type=jax.ShapeDtypeStruct((batch, hist, d), jnp.float32),
        scratch_types=[
            pltpu.VMEM((per_w, hist), jnp.int32),
            pltpu.VMEM((n_sem, hist, d), jnp.float32),
        ]
        + [pltpu.SemaphoreType.DMA] * (2 * n_sem),
    )
    def gather_kernel(table_hbm, idx_hbm, out_hbm, idx_v, rows_v, *sems):
        gsems = sems[:n_sem]
        wsems = sems[n_sem:]
        wid = lax.axis_index("s") * _NC + lax.axis_index("c")
        base = wid * per_w
        pltpu.sync_copy(idx_hbm.at[wid], idx_v)

        def g_src(i):
            return table_hbm.at[idx_v.at[i]]

        def out_dst(i):
            return out_hbm.at[base + i]

        def g_dst(s):
            return rows_v.at[s]

        for b in range(_RING):
            pltpu.async_copy(g_src(b), g_dst(b), gsems[b])

        def outer(gg, carry):
            for p in range(2):
                g = 2 * gg + p
                for b in range(_RING):
                    s = p * _RING + b
                    sn = (1 - p) * _RING + b
                    i = g * _RING + b
                    pltpu.make_async_copy(g_src(i), g_dst(s), gsems[s]).wait()
                    pltpu.async_copy(rows_v.at[s], out_dst(i), wsems[s])

                    def prefetch(i=i, sn=sn):
                        pltpu.make_async_copy(
                            rows_v.at[sn], out_dst(i - _RING), wsems[sn]
                        ).wait()
                        pltpu.async_copy(g_src(i + _RING), g_dst(sn), gsems[sn])

                    if p == 0:
                        @pl.when(gg > 0)
                        def _(i=i, sn=sn):
                            pltpu.make_async_copy(
                                rows_v.at[sn], out_dst(i - _RING), wsems[sn]
                            ).wait()

                        pltpu.async_copy(g_src(i + _RING), g_dst(sn), gsems[sn])
                    else:
                        pl.when(gg < half - 1)(prefetch)
            return carry

        lax.fori_loop(0, half, outer, 0)

        for b in range(_RING):
            i0 = (n_groups - 2) * _RING + b
            i1 = (n_groups - 1) * _RING + b
            pltpu.make_async_copy(rows_v.at[b], out_dst(i0), wsems[b]).wait()
            pltpu.make_async_copy(
                rows_v.at[_RING + b], out_dst(i1), wsems[_RING + b]
            ).wait()

    return gather_kernel


@jax.jit
def kernel(x, emb_matrix):
    b, h = x.shape
    vocab, d = emb_matrix.shape
    idx = x.reshape(_NW, b // _NW, h)
    return _build_gather(b, h, vocab, d)(emb_matrix, idx)
